# baseline (speedup 1.0000x reference)
import functools

import jax
import jax.numpy as jnp
import numpy as np
from jax.experimental import pallas as pl
from jax.experimental.pallas import tpu as pltpu
from jax.sharding import Mesh, PartitionSpec as P

try:
    from jax.experimental.shard_map import shard_map as _shard_map
except ImportError:
    _shard_map = jax.shard_map

BM = 1024
BN = 512


def _linear_kernel(x_ref, w_ref, b_ref, o_ref):
    x = x_ref[...].astype(jnp.bfloat16)
    w = w_ref[...].astype(jnp.bfloat16)
    acc = jnp.dot(x, w, preferred_element_type=jnp.float32)
    o_ref[...] = acc + b_ref[...]


def _linear_one_device(input, W, b2):
    m, k = input.shape
    _, n = W.shape
    bm = min(BM, m)
    grid = (m // bm, n // BN)
    return pl.pallas_call(
        _linear_kernel,
        grid=grid,
        in_specs=[
            pl.BlockSpec((bm, k), lambda i, j: (i, 0)),
            pl.BlockSpec((k, BN), lambda i, j: (0, j)),
            pl.BlockSpec((1, BN), lambda i, j: (0, j)),
        ],
        out_specs=pl.BlockSpec((bm, BN), lambda i, j: (i, j)),
        out_shape=jax.ShapeDtypeStruct((m, n), jnp.float32),
        compiler_params=pltpu.CompilerParams(
            dimension_semantics=("arbitrary", "arbitrary"),
        ),
    )(input, W, b2)


@functools.partial(jax.jit, static_argnames=())
def kernel(input, W, b):
    m, _ = input.shape
    n = W.shape[1]
    b2 = b.reshape(1, n)
    devs = jax.devices()
    nd = 1
    for cand in (4, 2):
        if len(devs) >= cand and m % (cand * 256) == 0:
            nd = cand
            break
    if nd == 1:
        return _linear_one_device(input, W, b2)
    mesh = Mesh(np.array(devs[:nd]), ("d",))
    f = _shard_map(
        _linear_one_device,
        mesh=mesh,
        in_specs=(P("d", None), P(None, None), P(None, None)),
        out_specs=P("d", None),
        check_rep=False,
    )
    return f(input, W, b2)


# speedup vs baseline: 2.0736x; 2.0736x over previous
import functools

import jax
import jax.numpy as jnp
from jax.experimental import pallas as pl
from jax.experimental.pallas import tpu as pltpu

BM = 1024
BN = 512


def _linear_kernel(x_hbm, w_ref, b_ref, o_ref, xbuf_ref, sems):
    i = pl.program_id(0)
    j = pl.program_id(1)
    ni = pl.num_programs(0)

    @pl.when(jnp.logical_and(i == 0, j == 0))
    def _start_first_strip():
        pltpu.make_async_copy(
            x_hbm.at[pl.ds(0, BM)], xbuf_ref.at[0], sems.at[0]
        ).start()

    @pl.when(j == 0)
    def _rotate_strips():
        pltpu.make_async_copy(
            x_hbm.at[pl.ds(i * BM, BM)], xbuf_ref.at[i % 2], sems.at[i % 2]
        ).wait()

        @pl.when(i + 1 < ni)
        def _start_next_strip():
            pltpu.make_async_copy(
                x_hbm.at[pl.ds((i + 1) * BM, BM)],
                xbuf_ref.at[(i + 1) % 2],
                sems.at[(i + 1) % 2],
            ).start()

    x = xbuf_ref[i % 2].astype(jnp.bfloat16)
    w = w_ref[...].astype(jnp.bfloat16)
    acc = jnp.dot(x, w, preferred_element_type=jnp.float32)
    o_ref[...] = acc + b_ref[...]


@functools.partial(jax.jit, static_argnames=())
def kernel(input, W, b):
    m, k = input.shape
    _, n = W.shape
    b2 = b.reshape(1, n)
    grid = (m // BM, n // BN)
    return pl.pallas_call(
        _linear_kernel,
        grid=grid,
        in_specs=[
            pl.BlockSpec(memory_space=pl.ANY),
            pl.BlockSpec((k, BN), lambda i, j: (0, j)),
            pl.BlockSpec((1, BN), lambda i, j: (0, j)),
        ],
        out_specs=pl.BlockSpec((BM, BN), lambda i, j: (i, j)),
        out_shape=jax.ShapeDtypeStruct((m, n), jnp.float32),
        scratch_shapes=[
            pltpu.VMEM((2, BM, k), jnp.float32),
            pltpu.SemaphoreType.DMA((2,)),
        ],
        compiler_params=pltpu.CompilerParams(
            dimension_semantics=("arbitrary", "arbitrary"),
        ),
    )(input, W, b2)


# speedup vs baseline: 2.2000x; 1.0610x over previous
import functools

import jax
import jax.numpy as jnp
from jax.experimental import pallas as pl
from jax.experimental.pallas import tpu as pltpu

BM = 1024
BN = 512


NCHUNK = 4


def _linear_kernel(x_hbm, w_ref, b_ref, o_ref, xbuf_ref, sems):
    i = pl.program_id(0)
    j = pl.program_id(1)
    ni = pl.num_programs(0)
    ch = BM // NCHUNK

    def _chunk_copy(strip, c):
        slot = jax.lax.rem(strip, 2)
        return pltpu.make_async_copy(
            x_hbm.at[pl.ds(strip * BM + c * ch, ch)],
            xbuf_ref.at[slot, pl.ds(c * ch, ch)],
            sems.at[slot, c],
        )

    @pl.when(jnp.logical_and(i == 0, j == 0))
    def _start_first_strip():
        for c in range(NCHUNK):
            _chunk_copy(0, c).start()

    @pl.when(j == 0)
    def _wait_strip():
        for c in range(NCHUNK):
            _chunk_copy(i, c).wait()

    @pl.when(jnp.logical_and(i + 1 < ni, jnp.logical_and(1 <= j, j <= NCHUNK)))
    def _start_next_chunk():
        _chunk_copy(i + 1, j - 1).start()

    x = xbuf_ref[i % 2].astype(jnp.bfloat16)
    w = w_ref[...].astype(jnp.bfloat16)
    acc = jnp.dot(x, w, preferred_element_type=jnp.float32)
    o_ref[...] = acc + b_ref[...]


@functools.partial(jax.jit, static_argnames=())
def kernel(input, W, b):
    m, k = input.shape
    _, n = W.shape
    b2 = b.reshape(1, n)
    grid = (m // BM, n // BN)
    return pl.pallas_call(
        _linear_kernel,
        grid=grid,
        in_specs=[
            pl.BlockSpec(memory_space=pl.ANY),
            pl.BlockSpec((k, BN), lambda i, j: (0, j)),
            pl.BlockSpec((1, BN), lambda i, j: (0, j)),
        ],
        out_specs=pl.BlockSpec((BM, BN), lambda i, j: (i, j)),
        out_shape=jax.ShapeDtypeStruct((m, n), jnp.float32),
        scratch_shapes=[
            pltpu.VMEM((2, BM, k), jnp.float32),
            pltpu.SemaphoreType.DMA((2, NCHUNK)),
        ],
        compiler_params=pltpu.CompilerParams(
            dimension_semantics=("arbitrary", "arbitrary"),
        ),
    )(input, W, b2)
